# baseline (speedup 1.0000x reference)
import functools

import jax
import jax.numpy as jnp
from jax import lax
from jax.experimental import pallas as pl
from jax.experimental.pallas import tpu as pltpu
from jax.experimental.pallas import tpu_sc as plsc

_L = 16
_NB = 160
_SC_ROWS = 10240
_TC_KN = 1280


def _make_sc_scatter(n, n_pad, v):
    info = plsc.get_sparse_core_info()
    nc, ns = info.num_cores, info.num_subcores
    nw = nc * ns
    sl = n_pad // nw
    v_pad = -(-v // _L) * _L

    mesh = plsc.VectorSubcoreMesh(core_axis_name="c", subcore_axis_name="s")

    @functools.partial(
        pl.kernel,
        mesh=mesh,
        out_type=(
            jax.ShapeDtypeStruct((n_pad,), jnp.float32),
            jax.ShapeDtypeStruct((1, n_pad), jnp.float32),
        ),
        scratch_types=[
            pltpu.VMEM((v_pad + _L,), jnp.int32),
            pltpu.VMEM((v_pad + _L,), jnp.float32),
            pltpu.VMEM((1, sl), jnp.float32),
        ],
        compiler_params=pltpu.CompilerParams(needs_layout_passes=False),
    )
    def sc_scatter(idx_hbm, w_hbm, out1_hbm, out2_hbm, idx_v, w_v, acc_v):
        wid = lax.axis_index("s") * nc + lax.axis_index("c")
        base = wid * sl
        pltpu.sync_copy(idx_hbm, idx_v.at[pl.ds(0, v)])
        pltpu.sync_copy(w_hbm, w_v.at[pl.ds(0, v)])
        if v_pad > v:
            idx_v[pl.ds(v, _L)] = jnp.full((_L,), n, jnp.int32)
            w_v[pl.ds(v, _L)] = jnp.zeros((_L,), jnp.float32)

        zeros = jnp.zeros((_L,), jnp.float32)
        zrow = jnp.zeros((_L,), jnp.int32)

        def zero_body(i, carry):
            acc_v[0, pl.ds(i * _L, _L)] = zeros
            return carry

        lax.fori_loop(0, sl // _L, zero_body, 0)

        def scat_body(i, carry):
            iv = idx_v[pl.ds(i * _L, _L)]
            wv = w_v[pl.ds(i * _L, _L)]
            loc = iv - base
            m = (loc >= 0) & (loc < sl)
            loc = jnp.where(m, loc, 0)
            plsc.store_scatter(acc_v, [zrow, loc], wv, mask=m)
            return carry

        lax.fori_loop(0, v_pad // _L, scat_body, 0)

        pltpu.sync_copy(acc_v, out2_hbm.at[pl.ds(0, 1), pl.ds(base, sl)])
        pltpu.sync_copy(acc_v.at[0], out1_hbm.at[pl.ds(base, sl)])

    return sc_scatter


def _make_sc_matvec(b, lanes_w, n_pad):
    info = plsc.get_sparse_core_info()
    nc, ns = info.num_cores, info.num_subcores
    n_blocks = _SC_ROWS // _NB
    n_acc = lanes_w // _L

    mesh = plsc.VectorSubcoreMesh(core_axis_name="c", subcore_axis_name="s")

    @functools.partial(
        pl.kernel,
        mesh=mesh,
        out_type=jax.ShapeDtypeStruct((b,), jnp.float32),
        scratch_types=[
            pltpu.VMEM((_NB, 128), jnp.float32),
            pltpu.VMEM((_NB, 128), jnp.float32),
            pltpu.VMEM((_NB, 128), jnp.float32),
            pltpu.VMEM((_NB, 128), jnp.float32),
            pltpu.VMEM((_SC_ROWS,), jnp.float32),
            pltpu.VMEM((lanes_w,), jnp.float32),
            pltpu.VMEM((_L,), jnp.float32),
            pltpu.SemaphoreType.DMA,
            pltpu.SemaphoreType.DMA,
            pltpu.SemaphoreType.DMA,
            pltpu.SemaphoreType.DMA,
        ],
        compiler_params=pltpu.CompilerParams(needs_layout_passes=False),
    )
    def sc_matvec(
        yt_hbm, wf_hbm, bias_hbm, out_hbm,
        buf0, buf1, buf2, buf3, wf_v, o_v, bias_v, sem0, sem1, sem2, sem3,
    ):
        bufs = (buf0, buf1, buf2, buf3)
        sems = (sem0, sem1, sem2, sem3)
        wid = lax.axis_index("s") * nc + lax.axis_index("c")
        col0 = wid * lanes_w

        def dma(k, bslot):
            return pltpu.make_async_copy(
                yt_hbm.at[pl.ds(k * _NB, _NB), pl.ds(col0, lanes_w)],
                bufs[bslot],
                sems[bslot],
            )

        pltpu.sync_copy(bias_hbm, bias_v)
        pltpu.sync_copy(wf_hbm.at[pl.ds(0, _SC_ROWS)], wf_v)
        dma(0, 0).start()
        dma(1, 1).start()
        dma(2, 2).start()
        dma(3, 3).start()

        bias_vec = bias_v[...]
        acc0 = tuple(jnp.zeros((_L,), jnp.float32) for _ in range(n_acc))
        splat_idx = [jnp.full((_L,), l, jnp.int32) for l in range(_L)]

        def block_reduce(buf, k, accs):
            def inner(nc_i, a):
                wv = wf_v[pl.ds(k * _NB + nc_i * _L, _L)]
                for l in range(_L):
                    ws = jnp.take(wv, splat_idx[l])
                    row = nc_i * _L + l
                    a = tuple(
                        a[m] + buf[row, pl.ds(m * _L, _L)] * ws
                        for m in range(n_acc)
                    )
                return a

            return lax.fori_loop(0, _NB // _L, inner, accs)

        def outer(kk, accs):
            for bslot in range(4):
                k = kk * 4 + bslot
                ok = k < n_blocks

                @pl.when(ok)
                def _(bslot=bslot, k=k):
                    dma(k, bslot).wait()

                new = block_reduce(bufs[bslot], k, accs)
                accs = tuple(
                    jnp.where(ok, nn, aa) for nn, aa in zip(new, accs)
                )

                @pl.when(k + 4 < n_blocks)
                def _(bslot=bslot, k=k):
                    dma(k + 4, bslot).start()
            return accs

        accs = lax.fori_loop(0, -(-n_blocks // 4), outer, acc0)

        for m in range(n_acc):
            o_v[pl.ds(m * _L, _L)] = accs[m] + bias_vec

        pltpu.sync_copy(o_v, out_hbm.at[pl.ds(col0, lanes_w)])

    return sc_matvec


def _make_tc_body(n):
    def _tc_body(y_ref, w_ref, o_ref):
        i = pl.program_id(0)

        @pl.when(i == 0)
        def _():
            o_ref[...] = jnp.zeros(o_ref.shape, jnp.float32)

        rows = y_ref.shape[0]
        ridx = lax.broadcasted_iota(jnp.int32, (rows, 1), 0) + (
            _SC_ROWS + i * rows
        )
        y_m = jnp.where(ridx < n, y_ref[...], 0.0)
        o_ref[...] = o_ref[...] + jax.lax.dot_general(
            w_ref[...], y_m,
            (((1,), (0,)), ((), ())),
            precision=jax.lax.Precision.HIGHEST,
            preferred_element_type=jnp.float32,
        )

    return _tc_body


def _tc_matvec(yt, w_row, n_pad):
    nt, b = yt.shape
    grid = ((n_pad - _SC_ROWS) // _TC_KN,)
    return pl.pallas_call(
        _make_tc_body(nt),
        grid=grid,
        in_specs=[
            pl.BlockSpec((_TC_KN, b), lambda i: (_SC_ROWS // _TC_KN + i, 0)),
            pl.BlockSpec((1, _TC_KN), lambda i: (0, _SC_ROWS // _TC_KN + i)),
        ],
        out_specs=pl.BlockSpec((1, b), lambda i: (0, 0)),
        out_shape=jax.ShapeDtypeStruct((1, b), jnp.float32),
    )(yt, w_row)


def kernel(YhatFull, weights, bias, nodeOrder):
    b, n = YhatFull.shape
    v = nodeOrder.shape[0]
    lanes_w = b // 32
    n_pad = -(-n // _TC_KN) * _TC_KN

    yt = YhatFull.T
    wf_flat, wf_row = _make_sc_scatter(n, n_pad, v)(
        nodeOrder.astype(jnp.int32), weights
    )
    sc_out = _make_sc_matvec(b, lanes_w, n_pad)(
        yt, wf_flat, jnp.broadcast_to(bias.reshape(1), (_L,))
    )
    tc_out = _tc_matvec(yt, wf_row, n_pad)
    return (sc_out + tc_out.reshape(b)).reshape(b, 1)


# speedup vs baseline: 1.0293x; 1.0293x over previous
import functools

import jax
import jax.numpy as jnp
from jax import lax
from jax.experimental import pallas as pl
from jax.experimental.pallas import tpu as pltpu
from jax.experimental.pallas import tpu_sc as plsc

_L = 16
_NB = 160
_SC_ROWS = 10240
_TC_KN = 1280


def _make_sc_scatter(n, n_pad, v):
    info = plsc.get_sparse_core_info()
    nc, ns = info.num_cores, info.num_subcores
    nw = nc * ns
    sl = n_pad // nw
    v_pad = -(-v // _L) * _L

    mesh = plsc.VectorSubcoreMesh(core_axis_name="c", subcore_axis_name="s")

    @functools.partial(
        pl.kernel,
        mesh=mesh,
        out_type=(
            jax.ShapeDtypeStruct((n_pad,), jnp.float32),
            jax.ShapeDtypeStruct((1, n_pad), jnp.float32),
        ),
        scratch_types=[
            pltpu.VMEM((v_pad + _L,), jnp.int32),
            pltpu.VMEM((v_pad + _L,), jnp.float32),
            pltpu.VMEM((1, sl), jnp.float32),
        ],
        compiler_params=pltpu.CompilerParams(needs_layout_passes=False),
    )
    def sc_scatter(idx_hbm, w_hbm, out1_hbm, out2_hbm, idx_v, w_v, acc_v):
        wid = lax.axis_index("s") * nc + lax.axis_index("c")
        base = wid * sl
        pltpu.sync_copy(idx_hbm, idx_v.at[pl.ds(0, v)])
        pltpu.sync_copy(w_hbm, w_v.at[pl.ds(0, v)])
        if v_pad > v:
            idx_v[pl.ds(v, _L)] = jnp.full((_L,), n, jnp.int32)
            w_v[pl.ds(v, _L)] = jnp.zeros((_L,), jnp.float32)

        zeros = jnp.zeros((_L,), jnp.float32)
        zrow = jnp.zeros((_L,), jnp.int32)

        def zero_body(i, carry):
            acc_v[0, pl.ds(i * _L, _L)] = zeros
            return carry

        lax.fori_loop(0, sl // _L, zero_body, 0)

        def scat_body(i, carry):
            iv = idx_v[pl.ds(i * _L, _L)]
            wv = w_v[pl.ds(i * _L, _L)]
            loc = iv - base
            m = (loc >= 0) & (loc < sl)
            loc = jnp.where(m, loc, 0)
            plsc.store_scatter(acc_v, [zrow, loc], wv, mask=m)
            return carry

        lax.fori_loop(0, v_pad // _L, scat_body, 0)

        pltpu.sync_copy(acc_v, out2_hbm.at[pl.ds(0, 1), pl.ds(base, sl)])
        pltpu.sync_copy(acc_v.at[0], out1_hbm.at[pl.ds(base, sl)])

    return sc_scatter


def _make_sc_matvec(b, lanes_w, n_pad):
    info = plsc.get_sparse_core_info()
    nc, ns = info.num_cores, info.num_subcores
    n_blocks = _SC_ROWS // _NB
    n_acc = lanes_w // _L

    mesh = plsc.VectorSubcoreMesh(core_axis_name="c", subcore_axis_name="s")

    @functools.partial(
        pl.kernel,
        mesh=mesh,
        out_type=jax.ShapeDtypeStruct((b,), jnp.float32),
        scratch_types=[
            pltpu.VMEM((_NB, 128), jnp.float32),
            pltpu.VMEM((_NB, 128), jnp.float32),
            pltpu.VMEM((_NB, 128), jnp.float32),
            pltpu.VMEM((_SC_ROWS,), jnp.float32),
            pltpu.VMEM((lanes_w,), jnp.float32),
            pltpu.VMEM((_L,), jnp.float32),
            pltpu.SemaphoreType.DMA,
            pltpu.SemaphoreType.DMA,
            pltpu.SemaphoreType.DMA,
        ],
        compiler_params=pltpu.CompilerParams(needs_layout_passes=False),
    )
    def sc_matvec(
        yt_hbm, wf_hbm, bias_hbm, out_hbm,
        buf0, buf1, buf2, wf_v, o_v, bias_v, sem0, sem1, sem2,
    ):
        bufs = (buf0, buf1, buf2)
        sems = (sem0, sem1, sem2)
        wid = lax.axis_index("s") * nc + lax.axis_index("c")
        col0 = wid * lanes_w

        def dma(k, bslot):
            return pltpu.make_async_copy(
                yt_hbm.at[pl.ds(k * _NB, _NB), pl.ds(col0, lanes_w)],
                bufs[bslot],
                sems[bslot],
            )

        pltpu.sync_copy(bias_hbm, bias_v)
        pltpu.sync_copy(wf_hbm.at[pl.ds(0, _SC_ROWS)], wf_v)
        dma(0, 0).start()
        dma(1, 1).start()
        dma(2, 2).start()

        bias_vec = bias_v[...]
        acc0 = tuple(jnp.zeros((_L,), jnp.float32) for _ in range(n_acc))
        splat_idx = [jnp.full((_L,), l, jnp.int32) for l in range(_L)]

        def block_reduce(buf, k, accs):
            def inner(nc_i, a):
                wv = wf_v[pl.ds(k * _NB + nc_i * _L, _L)]
                for l in range(_L):
                    ws = jnp.take(wv, splat_idx[l])
                    row = nc_i * _L + l
                    a = tuple(
                        a[m] + buf[row, pl.ds(m * _L, _L)] * ws
                        for m in range(n_acc)
                    )
                return a

            return lax.fori_loop(0, _NB // _L, inner, accs)

        def outer(kk, accs):
            for bslot in range(3):
                k = kk * 3 + bslot
                ok = k < n_blocks

                @pl.when(ok)
                def _(bslot=bslot, k=k):
                    dma(k, bslot).wait()

                new = block_reduce(bufs[bslot], k, accs)
                accs = tuple(
                    jnp.where(ok, nn, aa) for nn, aa in zip(new, accs)
                )

                @pl.when(k + 3 < n_blocks)
                def _(bslot=bslot, k=k):
                    dma(k + 3, bslot).start()
            return accs

        accs = lax.fori_loop(0, -(-n_blocks // 3), outer, acc0)

        for m in range(n_acc):
            o_v[pl.ds(m * _L, _L)] = accs[m] + bias_vec

        pltpu.sync_copy(o_v, out_hbm.at[pl.ds(col0, lanes_w)])

    return sc_matvec


def _make_tc_body(n):
    def _tc_body(y_ref, w_ref, o_ref):
        i = pl.program_id(0)

        @pl.when(i == 0)
        def _():
            o_ref[...] = jnp.zeros(o_ref.shape, jnp.float32)

        rows = y_ref.shape[0]
        ridx = lax.broadcasted_iota(jnp.int32, (rows, 1), 0) + (
            _SC_ROWS + i * rows
        )
        y_m = jnp.where(ridx < n, y_ref[...], 0.0)
        o_ref[...] = o_ref[...] + jax.lax.dot_general(
            w_ref[...], y_m,
            (((1,), (0,)), ((), ())),
            precision=jax.lax.Precision.HIGHEST,
            preferred_element_type=jnp.float32,
        )

    return _tc_body


def _tc_matvec(yt, w_row, n_pad):
    nt, b = yt.shape
    grid = ((n_pad - _SC_ROWS) // _TC_KN,)
    return pl.pallas_call(
        _make_tc_body(nt),
        grid=grid,
        in_specs=[
            pl.BlockSpec((_TC_KN, b), lambda i: (_SC_ROWS // _TC_KN + i, 0)),
            pl.BlockSpec((1, _TC_KN), lambda i: (0, _SC_ROWS // _TC_KN + i)),
        ],
        out_specs=pl.BlockSpec((1, b), lambda i: (0, 0)),
        out_shape=jax.ShapeDtypeStruct((1, b), jnp.float32),
    )(yt, w_row)


def kernel(YhatFull, weights, bias, nodeOrder):
    b, n = YhatFull.shape
    v = nodeOrder.shape[0]
    lanes_w = b // 32
    n_pad = -(-n // _TC_KN) * _TC_KN

    yt = YhatFull.T
    wf_flat, wf_row = _make_sc_scatter(n, n_pad, v)(
        nodeOrder.astype(jnp.int32), weights
    )
    sc_out = _make_sc_matvec(b, lanes_w, n_pad)(
        yt, wf_flat, jnp.broadcast_to(bias.reshape(1), (_L,))
    )
    tc_out = _tc_matvec(yt, wf_row, n_pad)
    return (sc_out + tc_out.reshape(b)).reshape(b, 1)
